# baseline (speedup 1.0000x reference)
import functools

import jax
import jax.numpy as jnp
from jax import lax
from jax.experimental import pallas as pl
from jax.experimental.pallas import tpu as pltpu, tpu_sc as plsc

N = 10000
H = 128
L = 3

NC = 2
NS = 16
NW = NC * NS
K = 128
NBUF = 2

N_PAD = 10112
RPT = N_PAD // NS

ROW_BLK = 1000


def _sc_scatter_body(src_hbm, dst_hbm, m_hbm, out_hbm, src_v, dst_v, buf_v,
                     agg_sh, sem):
    c = lax.axis_index("c")
    s = lax.axis_index("s")
    w = c * NS + s
    n_chunks = src_hbm.shape[1]
    c2 = n_chunks // 2

    z = jnp.zeros((16,), jnp.float32)

    def zrow(i, carry):
        for j in range(H // 16):
            buf_v[0, i, pl.ds(j * 16, 16)] = z
        return carry

    lax.fori_loop(0, K, zrow, 0)
    r0 = s * RPT
    full, rem = RPT // K, RPT % K
    for t in range(full):
        pltpu.sync_copy(buf_v.at[0], agg_sh.at[pl.ds(r0 + t * K, K)])
    if rem:
        pltpu.sync_copy(buf_v.at[0, pl.ds(0, rem)],
                        agg_sh.at[pl.ds(r0 + full * K, rem)])
    plsc.subcore_barrier()

    def run_half(h0):
        pltpu.sync_copy(src_hbm.at[w, pl.ds(h0, c2)], src_v)
        pltpu.sync_copy(dst_hbm.at[w, pl.ds(h0, c2)], dst_v)

        def chunk(j, carry):
            pltpu.async_copy(m_hbm.at[src_v.at[j]], buf_v.at[0],
                             sem.at[0]).wait()
            pltpu.sync_copy(buf_v.at[0], agg_sh.at[dst_v.at[j]], add=True)
            return carry

        lax.fori_loop(0, c2, chunk, 0)

    run_half(0)
    run_half(c2)
    plsc.subcore_barrier()

    pltpu.sync_copy(agg_sh.at[pl.ds(r0, RPT)], out_hbm.at[c, pl.ds(r0, RPT)])


def _make_sc_scatter(n_chunks):
    mesh = plsc.VectorSubcoreMesh(core_axis_name="c", subcore_axis_name="s",
                                  num_cores=NC, num_subcores=NS)

    return pl.kernel(
        _sc_scatter_body,
        out_type=jax.ShapeDtypeStruct((NC, N_PAD, H), jnp.float32),
        mesh=mesh,
        scratch_types=[
            pltpu.VMEM((n_chunks // 2, K), jnp.int32),
            pltpu.VMEM((n_chunks // 2, K), jnp.int32),
            pltpu.VMEM((NBUF, K, H), jnp.float32),
            pltpu.VMEM_SHARED((N_PAD, H), jnp.float32),
            pltpu.SemaphoreType.DMA((NBUF,)),
        ],
    )


def _lin_msg_body(x_ref, lw_ref, lb_ref, w0_ref, h_ref, m_ref):
    h = jnp.dot(x_ref[...], lw_ref[...], preferred_element_type=jnp.float32)
    h = h + lb_ref[...]
    h_ref[...] = h
    m_ref[...] = jnp.dot(h, w0_ref[...], preferred_element_type=jnp.float32)


def _gru_core(parts_ref, h_ref, wih_ref, whh_ref, bih_ref, bhh_ref):
    agg = parts_ref[0] + parts_ref[1]
    h = h_ref[...]
    gi = jnp.dot(agg, wih_ref[...], preferred_element_type=jnp.float32)
    gi = gi + bih_ref[...]
    gh = jnp.dot(h, whh_ref[...], preferred_element_type=jnp.float32)
    gh = gh + bhh_ref[...]
    r = jax.nn.sigmoid(gi[:, :H] + gh[:, :H])
    zg = jax.nn.sigmoid(gi[:, H:2 * H] + gh[:, H:2 * H])
    n = jnp.tanh(gi[:, 2 * H:] + r * gh[:, 2 * H:])
    return (1.0 - zg) * n + zg * h


def _gru_msg_body(parts_ref, h_ref, wih_ref, whh_ref, bih_ref, bhh_ref,
                  wn_ref, hn_ref, mn_ref):
    hn = _gru_core(parts_ref, h_ref, wih_ref, whh_ref, bih_ref, bhh_ref)
    hn_ref[...] = hn
    mn_ref[...] = jnp.dot(hn, wn_ref[...], preferred_element_type=jnp.float32)


def _gru_out_body(parts_ref, h_ref, wih_ref, whh_ref, bih_ref, bhh_ref,
                  w2_ref, b2_ref, out_ref):
    hn = _gru_core(parts_ref, h_ref, wih_ref, whh_ref, bih_ref, bhh_ref)
    hn = jnp.maximum(hn, 0.0)
    out = jnp.dot(hn, w2_ref[...], preferred_element_type=jnp.float32)
    out_ref[...] = out + b2_ref[...]


def _row_spec(cols):
    return pl.BlockSpec((ROW_BLK, cols), lambda r: (r, 0))


def _full_spec(rows, cols):
    return pl.BlockSpec((rows, cols), lambda r: (0, 0))


def _parts_spec():
    return pl.BlockSpec((NC, ROW_BLK, H), lambda r: (0, r, 0))


_GRID = (N // ROW_BLK,)


@jax.jit
def _lin_msg(x, lin_W, lin_b, w0):
    return pl.pallas_call(
        _lin_msg_body,
        grid=_GRID,
        in_specs=[
            _row_spec(H),
            _full_spec(H, H),
            _full_spec(1, H),
            _full_spec(H, H),
        ],
        out_specs=[_row_spec(H), _row_spec(H)],
        out_shape=[
            jax.ShapeDtypeStruct((N, H), jnp.float32),
            jax.ShapeDtypeStruct((N, H), jnp.float32),
        ],
    )(x, lin_W, lin_b, w0)


@jax.jit
def _gru_msg(parts, h, wihT, whhT, bih, bhh, wn):
    return pl.pallas_call(
        _gru_msg_body,
        grid=_GRID,
        in_specs=[
            _parts_spec(),
            _row_spec(H),
            _full_spec(H, 3 * H),
            _full_spec(H, 3 * H),
            _full_spec(1, 3 * H),
            _full_spec(1, 3 * H),
            _full_spec(H, H),
        ],
        out_specs=[_row_spec(H), _row_spec(H)],
        out_shape=[
            jax.ShapeDtypeStruct((N, H), jnp.float32),
            jax.ShapeDtypeStruct((N, H), jnp.float32),
        ],
    )(parts, h, wihT, whhT, bih, bhh, wn)


@jax.jit
def _gru_out(parts, h, wihT, whhT, bih, bhh, w2, b2):
    return pl.pallas_call(
        _gru_out_body,
        grid=_GRID,
        in_specs=[
            _parts_spec(),
            _row_spec(H),
            _full_spec(H, 3 * H),
            _full_spec(H, 3 * H),
            _full_spec(1, 3 * H),
            _full_spec(1, 3 * H),
            _full_spec(H, H),
            _full_spec(1, H),
        ],
        out_specs=_row_spec(H),
        out_shape=jax.ShapeDtypeStruct((N, H), jnp.float32),
    )(parts, h, wihT, whhT, bih, bhh, w2, b2)


def kernel(x, edge_index, lin_W, lin_b, ggc_weight, W_ih, W_hh, b_ih, b_hh,
           lin2_W, lin2_b):
    E = edge_index.shape[1]
    n_chunks = -(-E // (NW * K))
    n_chunks = -(-n_chunks // (2 * NBUF)) * (2 * NBUF)
    e_pad = NW * n_chunks * K

    src = edge_index[0].astype(jnp.int32)
    dst = edge_index[1].astype(jnp.int32)
    if e_pad != E:
        pad = e_pad - E
        src = jnp.concatenate([src, jnp.zeros((pad,), jnp.int32)])
        dst = jnp.concatenate([dst, jnp.full((pad,), N, jnp.int32)])
    src3 = src.reshape(NW, n_chunks, K)
    dst3 = dst.reshape(NW, n_chunks, K)

    sc_scatter = _make_sc_scatter(n_chunks)

    wihT = W_ih.T
    whhT = W_hh.T
    bih = b_ih.reshape(1, 3 * H)
    bhh = b_hh.reshape(1, 3 * H)
    lb = lin_b.reshape(1, H)
    b2 = lin2_b.reshape(1, H)

    h, m = _lin_msg(x, lin_W, lb, ggc_weight[0])
    for i in range(L):
        parts = sc_scatter(src3, dst3, m)
        if i < L - 1:
            h, m = _gru_msg(parts, h, wihT, whhT, bih, bhh, ggc_weight[i + 1])
        else:
            out = _gru_out(parts, h, wihT, whhT, bih, bhh, lin2_W, b2)
    return out


# speedup vs baseline: 1.0205x; 1.0205x over previous
import functools

import jax
import jax.numpy as jnp
from jax import lax
from jax.experimental import pallas as pl
from jax.experimental.pallas import tpu as pltpu, tpu_sc as plsc

N = 10000
H = 128
L = 3

NC = 2
NS = 16
NW = NC * NS
K = 128
NBUF = 2

N_PAD = 10112
RPT = N_PAD // NS

ROW_BLK = 1000


def _sc_scatter_body(src_hbm, dst_hbm, m_hbm, out_hbm, src_v, dst_v, buf_a,
                     buf_b, agg_sh, sem_a, sem_b):
    c = lax.axis_index("c")
    s = lax.axis_index("s")
    w = c * NS + s
    n_chunks = src_hbm.shape[1]
    c2 = n_chunks // 2

    z = jnp.zeros((16,), jnp.float32)

    def zrow(i, carry):
        for j in range(H // 16):
            buf_a[i, pl.ds(j * 16, 16)] = z
        return carry

    lax.fori_loop(0, K, zrow, 0)
    r0 = s * RPT
    full, rem = RPT // K, RPT % K
    for t in range(full):
        pltpu.sync_copy(buf_a, agg_sh.at[pl.ds(r0 + t * K, K)])
    if rem:
        pltpu.sync_copy(buf_a.at[pl.ds(0, rem)],
                        agg_sh.at[pl.ds(r0 + full * K, rem)])
    plsc.subcore_barrier()

    def run_half(h0):
        pltpu.sync_copy(src_hbm.at[w, pl.ds(h0, c2)], src_v)
        pltpu.sync_copy(dst_hbm.at[w, pl.ds(h0, c2)], dst_v)

        def pair(g, carry):
            j0 = 2 * g
            j1 = 2 * g + 1
            da = pltpu.async_copy(m_hbm.at[src_v.at[j0]], buf_a, sem_a)
            db = pltpu.async_copy(m_hbm.at[src_v.at[j1]], buf_b, sem_b)
            da.wait()
            pltpu.sync_copy(buf_a, agg_sh.at[dst_v.at[j0]], add=True)
            db.wait()
            pltpu.sync_copy(buf_b, agg_sh.at[dst_v.at[j1]], add=True)
            return carry

        lax.fori_loop(0, c2 // 2, pair, 0)

    run_half(0)
    run_half(c2)
    plsc.subcore_barrier()

    pltpu.sync_copy(agg_sh.at[pl.ds(r0, RPT)], out_hbm.at[c, pl.ds(r0, RPT)])


def _make_sc_scatter(n_chunks):
    mesh = plsc.VectorSubcoreMesh(core_axis_name="c", subcore_axis_name="s",
                                  num_cores=NC, num_subcores=NS)

    return pl.kernel(
        _sc_scatter_body,
        out_type=jax.ShapeDtypeStruct((NC, N_PAD, H), jnp.float32),
        mesh=mesh,
        scratch_types=[
            pltpu.VMEM((n_chunks // 2, K), jnp.int32),
            pltpu.VMEM((n_chunks // 2, K), jnp.int32),
            pltpu.VMEM((K, H), jnp.float32),
            pltpu.VMEM((K, H), jnp.float32),
            pltpu.VMEM_SHARED((N_PAD, H), jnp.float32),
            pltpu.SemaphoreType.DMA,
            pltpu.SemaphoreType.DMA,
        ],
    )


def _lin_msg_body(x_ref, lw_ref, lb_ref, w0_ref, h_ref, m_ref):
    h = jnp.dot(x_ref[...], lw_ref[...], preferred_element_type=jnp.float32)
    h = h + lb_ref[...]
    h_ref[...] = h
    m_ref[...] = jnp.dot(h, w0_ref[...], preferred_element_type=jnp.float32)


def _gru_core(parts_ref, h_ref, wih_ref, whh_ref, bih_ref, bhh_ref):
    agg = parts_ref[0] + parts_ref[1]
    h = h_ref[...]
    gi = jnp.dot(agg, wih_ref[...], preferred_element_type=jnp.float32)
    gi = gi + bih_ref[...]
    gh = jnp.dot(h, whh_ref[...], preferred_element_type=jnp.float32)
    gh = gh + bhh_ref[...]
    r = jax.nn.sigmoid(gi[:, :H] + gh[:, :H])
    zg = jax.nn.sigmoid(gi[:, H:2 * H] + gh[:, H:2 * H])
    n = jnp.tanh(gi[:, 2 * H:] + r * gh[:, 2 * H:])
    return (1.0 - zg) * n + zg * h


def _gru_msg_body(parts_ref, h_ref, wih_ref, whh_ref, bih_ref, bhh_ref,
                  wn_ref, hn_ref, mn_ref):
    hn = _gru_core(parts_ref, h_ref, wih_ref, whh_ref, bih_ref, bhh_ref)
    hn_ref[...] = hn
    mn_ref[...] = jnp.dot(hn, wn_ref[...], preferred_element_type=jnp.float32)


def _gru_out_body(parts_ref, h_ref, wih_ref, whh_ref, bih_ref, bhh_ref,
                  w2_ref, b2_ref, out_ref):
    hn = _gru_core(parts_ref, h_ref, wih_ref, whh_ref, bih_ref, bhh_ref)
    hn = jnp.maximum(hn, 0.0)
    out = jnp.dot(hn, w2_ref[...], preferred_element_type=jnp.float32)
    out_ref[...] = out + b2_ref[...]


def _row_spec(cols):
    return pl.BlockSpec((ROW_BLK, cols), lambda r: (r, 0))


def _full_spec(rows, cols):
    return pl.BlockSpec((rows, cols), lambda r: (0, 0))


def _parts_spec():
    return pl.BlockSpec((NC, ROW_BLK, H), lambda r: (0, r, 0))


_GRID = (N // ROW_BLK,)


@jax.jit
def _lin_msg(x, lin_W, lin_b, w0):
    return pl.pallas_call(
        _lin_msg_body,
        grid=_GRID,
        in_specs=[
            _row_spec(H),
            _full_spec(H, H),
            _full_spec(1, H),
            _full_spec(H, H),
        ],
        out_specs=[_row_spec(H), _row_spec(H)],
        out_shape=[
            jax.ShapeDtypeStruct((N, H), jnp.float32),
            jax.ShapeDtypeStruct((N, H), jnp.float32),
        ],
    )(x, lin_W, lin_b, w0)


@jax.jit
def _gru_msg(parts, h, wihT, whhT, bih, bhh, wn):
    return pl.pallas_call(
        _gru_msg_body,
        grid=_GRID,
        in_specs=[
            _parts_spec(),
            _row_spec(H),
            _full_spec(H, 3 * H),
            _full_spec(H, 3 * H),
            _full_spec(1, 3 * H),
            _full_spec(1, 3 * H),
            _full_spec(H, H),
        ],
        out_specs=[_row_spec(H), _row_spec(H)],
        out_shape=[
            jax.ShapeDtypeStruct((N, H), jnp.float32),
            jax.ShapeDtypeStruct((N, H), jnp.float32),
        ],
    )(parts, h, wihT, whhT, bih, bhh, wn)


@jax.jit
def _gru_out(parts, h, wihT, whhT, bih, bhh, w2, b2):
    return pl.pallas_call(
        _gru_out_body,
        grid=_GRID,
        in_specs=[
            _parts_spec(),
            _row_spec(H),
            _full_spec(H, 3 * H),
            _full_spec(H, 3 * H),
            _full_spec(1, 3 * H),
            _full_spec(1, 3 * H),
            _full_spec(H, H),
            _full_spec(1, H),
        ],
        out_specs=_row_spec(H),
        out_shape=jax.ShapeDtypeStruct((N, H), jnp.float32),
    )(parts, h, wihT, whhT, bih, bhh, w2, b2)


def kernel(x, edge_index, lin_W, lin_b, ggc_weight, W_ih, W_hh, b_ih, b_hh,
           lin2_W, lin2_b):
    E = edge_index.shape[1]
    n_chunks = -(-E // (NW * K))
    n_chunks = -(-n_chunks // (2 * NBUF)) * (2 * NBUF)
    e_pad = NW * n_chunks * K

    src = edge_index[0].astype(jnp.int32)
    dst = edge_index[1].astype(jnp.int32)
    if e_pad != E:
        pad = e_pad - E
        src = jnp.concatenate([src, jnp.zeros((pad,), jnp.int32)])
        dst = jnp.concatenate([dst, jnp.full((pad,), N, jnp.int32)])
    src3 = src.reshape(NW, n_chunks, K)
    dst3 = dst.reshape(NW, n_chunks, K)

    sc_scatter = _make_sc_scatter(n_chunks)

    wihT = W_ih.T
    whhT = W_hh.T
    bih = b_ih.reshape(1, 3 * H)
    bhh = b_hh.reshape(1, 3 * H)
    lb = lin_b.reshape(1, H)
    b2 = lin2_b.reshape(1, H)

    h, m = _lin_msg(x, lin_W, lb, ggc_weight[0])
    for i in range(L):
        parts = sc_scatter(src3, dst3, m)
        if i < L - 1:
            h, m = _gru_msg(parts, h, wihT, whhT, bih, bhh, ggc_weight[i + 1])
        else:
            out = _gru_out(parts, h, wihT, whhT, bih, bhh, lin2_W, b2)
    return out


# speedup vs baseline: 1.5325x; 1.5018x over previous
import functools

import jax
import jax.numpy as jnp
from jax import lax
from jax.experimental import pallas as pl
from jax.experimental.pallas import tpu as pltpu, tpu_sc as plsc

N = 10000
H = 128
L = 3

NC = 2
NS = 16
NW = NC * NS
K = 128
NBUF = 2

N_PAD = 10112
RPT = N_PAD // NS

ROW_BLK = 1000


def _sc_scatter_body(src_hbm, dst_hbm, m_hbm, out_hbm, src_v, dst_v, buf_a,
                     agg_sh, sem_a):
    c = lax.axis_index("c")
    s = lax.axis_index("s")
    w = c * NS + s
    n_chunks = src_hbm.shape[1]

    pltpu.sync_copy(src_hbm.at[w], src_v)
    pltpu.sync_copy(dst_hbm.at[w], dst_v)

    z = jnp.zeros((16,), jnp.float32)

    def zrow(i, carry):
        for j in range(H // 16):
            buf_a[i, pl.ds(j * 16, 16)] = z
        return carry

    lax.fori_loop(0, K, zrow, 0)
    r0 = s * RPT
    full, rem = RPT // K, RPT % K
    for t in range(full):
        pltpu.sync_copy(buf_a, agg_sh.at[pl.ds(r0 + t * K, K)])
    if rem:
        pltpu.sync_copy(buf_a.at[pl.ds(0, rem)],
                        agg_sh.at[pl.ds(r0 + full * K, rem)])
    plsc.subcore_barrier()

    def chunk(j, carry):
        pltpu.async_copy(m_hbm.at[src_v.at[j]], buf_a, sem_a).wait()
        pltpu.sync_copy(buf_a, agg_sh.at[dst_v.at[j]], add=True)
        return carry

    lax.fori_loop(0, n_chunks, chunk, 0)
    plsc.subcore_barrier()

    pltpu.sync_copy(agg_sh.at[pl.ds(r0, RPT)], out_hbm.at[c, pl.ds(r0, RPT)])


def _make_sc_scatter(n_chunks):
    mesh = plsc.VectorSubcoreMesh(core_axis_name="c", subcore_axis_name="s",
                                  num_cores=NC, num_subcores=NS)

    return pl.kernel(
        _sc_scatter_body,
        out_type=jax.ShapeDtypeStruct((NC, N_PAD, H), jnp.float32),
        mesh=mesh,
        scratch_types=[
            pltpu.VMEM((n_chunks, K), jnp.int32),
            pltpu.VMEM((n_chunks, K), jnp.int32),
            pltpu.VMEM((K, H), jnp.float32),
            pltpu.VMEM_SHARED((N_PAD, H), jnp.float32),
            pltpu.SemaphoreType.DMA,
        ],
    )


def _lin_msg_body(x_ref, lw_ref, lb_ref, w0_ref, h_ref, m_ref):
    h = jnp.dot(x_ref[...], lw_ref[...], preferred_element_type=jnp.float32)
    h = h + lb_ref[...]
    h_ref[...] = h
    m_ref[...] = jnp.dot(h, w0_ref[...], preferred_element_type=jnp.float32)


def _gru_core(parts_ref, h_ref, wih_ref, whh_ref, bih_ref, bhh_ref):
    agg = parts_ref[0] + parts_ref[1]
    h = h_ref[...]
    gi = jnp.dot(agg, wih_ref[...], preferred_element_type=jnp.float32)
    gi = gi + bih_ref[...]
    gh = jnp.dot(h, whh_ref[...], preferred_element_type=jnp.float32)
    gh = gh + bhh_ref[...]
    r = jax.nn.sigmoid(gi[:, :H] + gh[:, :H])
    zg = jax.nn.sigmoid(gi[:, H:2 * H] + gh[:, H:2 * H])
    n = jnp.tanh(gi[:, 2 * H:] + r * gh[:, 2 * H:])
    return (1.0 - zg) * n + zg * h


def _gru_msg_body(parts_ref, h_ref, wih_ref, whh_ref, bih_ref, bhh_ref,
                  wn_ref, hn_ref, mn_ref):
    hn = _gru_core(parts_ref, h_ref, wih_ref, whh_ref, bih_ref, bhh_ref)
    hn_ref[...] = hn
    mn_ref[...] = jnp.dot(hn, wn_ref[...], preferred_element_type=jnp.float32)


def _gru_out_body(parts_ref, h_ref, wih_ref, whh_ref, bih_ref, bhh_ref,
                  w2_ref, b2_ref, out_ref):
    hn = _gru_core(parts_ref, h_ref, wih_ref, whh_ref, bih_ref, bhh_ref)
    hn = jnp.maximum(hn, 0.0)
    out = jnp.dot(hn, w2_ref[...], preferred_element_type=jnp.float32)
    out_ref[...] = out + b2_ref[...]


def _row_spec(cols):
    return pl.BlockSpec((ROW_BLK, cols), lambda r: (r, 0))


def _full_spec(rows, cols):
    return pl.BlockSpec((rows, cols), lambda r: (0, 0))


def _parts_spec():
    return pl.BlockSpec((NC, ROW_BLK, H), lambda r: (0, r, 0))


_GRID = (N // ROW_BLK,)


@jax.jit
def _lin_msg(x, lin_W, lin_b, w0):
    return pl.pallas_call(
        _lin_msg_body,
        grid=_GRID,
        in_specs=[
            _row_spec(H),
            _full_spec(H, H),
            _full_spec(1, H),
            _full_spec(H, H),
        ],
        out_specs=[_row_spec(H), _row_spec(H)],
        out_shape=[
            jax.ShapeDtypeStruct((N, H), jnp.float32),
            jax.ShapeDtypeStruct((N, H), jnp.float32),
        ],
    )(x, lin_W, lin_b, w0)


@jax.jit
def _gru_msg(parts, h, wihT, whhT, bih, bhh, wn):
    return pl.pallas_call(
        _gru_msg_body,
        grid=_GRID,
        in_specs=[
            _parts_spec(),
            _row_spec(H),
            _full_spec(H, 3 * H),
            _full_spec(H, 3 * H),
            _full_spec(1, 3 * H),
            _full_spec(1, 3 * H),
            _full_spec(H, H),
        ],
        out_specs=[_row_spec(H), _row_spec(H)],
        out_shape=[
            jax.ShapeDtypeStruct((N, H), jnp.float32),
            jax.ShapeDtypeStruct((N, H), jnp.float32),
        ],
    )(parts, h, wihT, whhT, bih, bhh, wn)


@jax.jit
def _gru_out(parts, h, wihT, whhT, bih, bhh, w2, b2):
    return pl.pallas_call(
        _gru_out_body,
        grid=_GRID,
        in_specs=[
            _parts_spec(),
            _row_spec(H),
            _full_spec(H, 3 * H),
            _full_spec(H, 3 * H),
            _full_spec(1, 3 * H),
            _full_spec(1, 3 * H),
            _full_spec(H, H),
            _full_spec(1, H),
        ],
        out_specs=_row_spec(H),
        out_shape=jax.ShapeDtypeStruct((N, H), jnp.float32),
    )(parts, h, wihT, whhT, bih, bhh, w2, b2)


def kernel(x, edge_index, lin_W, lin_b, ggc_weight, W_ih, W_hh, b_ih, b_hh,
           lin2_W, lin2_b):
    E = edge_index.shape[1]
    n_chunks = -(-E // (NW * K))
    e_pad = NW * n_chunks * K

    src = edge_index[0].astype(jnp.int32)
    dst = edge_index[1].astype(jnp.int32)
    if e_pad != E:
        pad = e_pad - E
        src = jnp.concatenate([src, jnp.zeros((pad,), jnp.int32)])
        dst = jnp.concatenate([dst, jnp.full((pad,), N, jnp.int32)])
    src3 = src.reshape(NW, n_chunks, K)
    dst3 = dst.reshape(NW, n_chunks, K)

    sc_scatter = _make_sc_scatter(n_chunks)

    wihT = W_ih.T
    whhT = W_hh.T
    bih = b_ih.reshape(1, 3 * H)
    bhh = b_hh.reshape(1, 3 * H)
    lb = lin_b.reshape(1, H)
    b2 = lin2_b.reshape(1, H)

    h, m = _lin_msg(x, lin_W, lb, ggc_weight[0])
    for i in range(L):
        parts = sc_scatter(src3, dst3, m)
        if i < L - 1:
            h, m = _gru_msg(parts, h, wihT, whhT, bih, bhh, ggc_weight[i + 1])
        else:
            out = _gru_out(parts, h, wihT, whhT, bih, bhh, lin2_W, b2)
    return out


# speedup vs baseline: 2.8493x; 1.8592x over previous
import functools

import jax
import jax.numpy as jnp
from jax import lax
from jax.experimental import pallas as pl
from jax.experimental.pallas import tpu as pltpu, tpu_sc as plsc

N = 10000
H = 128
L = 3

NC = 2
NS = 16
NW = NC * NS
K = 128
NBUF = 2

N_PAD = 10112
RPT = N_PAD // NS

ROW_BLK = 1000


def _sc_scatter_body(src_hbm, dst_hbm, m_hbm, out_hbm, src_v, dst_v, buf_a,
                     agg_sh, sem_a):
    c = lax.axis_index("c")
    s = lax.axis_index("s")
    w = c * NS + s
    n_chunks = src_hbm.shape[1]

    pltpu.sync_copy(src_hbm.at[w], src_v)
    pltpu.sync_copy(dst_hbm.at[w], dst_v)

    z = jnp.zeros((16,), jnp.float32)

    def zrow(i, carry):
        for j in range(H // 16):
            buf_a[i, pl.ds(j * 16, 16)] = z
        return carry

    lax.fori_loop(0, K, zrow, 0)
    r0 = s * RPT
    full, rem = RPT // K, RPT % K
    for t in range(full):
        pltpu.sync_copy(buf_a, agg_sh.at[pl.ds(r0 + t * K, K)])
    if rem:
        pltpu.sync_copy(buf_a.at[pl.ds(0, rem)],
                        agg_sh.at[pl.ds(r0 + full * K, rem)])
    plsc.subcore_barrier()

    def chunk(j, carry):
        pltpu.async_copy(m_hbm.at[src_v.at[j]], buf_a, sem_a).wait()
        pltpu.sync_copy(buf_a, agg_sh.at[dst_v.at[j]], add=True)
        return carry

    lax.fori_loop(0, n_chunks, chunk, 0)
    plsc.subcore_barrier()

    pltpu.sync_copy(agg_sh.at[pl.ds(r0, RPT)], out_hbm.at[c, pl.ds(r0, RPT)])


def _make_sc_scatter(n_chunks):
    mesh = plsc.VectorSubcoreMesh(core_axis_name="c", subcore_axis_name="s",
                                  num_cores=NC, num_subcores=NS)

    return pl.kernel(
        _sc_scatter_body,
        out_type=jax.ShapeDtypeStruct((NC, N_PAD, H), jnp.float32),
        mesh=mesh,
        scratch_types=[
            pltpu.VMEM((n_chunks, K), jnp.int32),
            pltpu.VMEM((n_chunks, K), jnp.int32),
            pltpu.VMEM((K, H), jnp.float32),
            pltpu.VMEM_SHARED((N_PAD, H), jnp.float32),
            pltpu.SemaphoreType.DMA,
        ],
    )


def _lin_msg_body(x_ref, lw_ref, lb_ref, w0_ref, h_ref, m_ref):
    h = jnp.dot(x_ref[...], lw_ref[...], preferred_element_type=jnp.float32)
    h = h + lb_ref[...]
    h_ref[...] = h
    m_ref[...] = jnp.dot(h, w0_ref[...], preferred_element_type=jnp.float32)


def _gru_core(parts_ref, h_ref, wih_ref, whh_ref, bih_ref, bhh_ref):
    agg = parts_ref[0] + parts_ref[1]
    h = h_ref[...]
    gi = jnp.dot(agg, wih_ref[...], preferred_element_type=jnp.float32)
    gi = gi + bih_ref[...]
    gh = jnp.dot(h, whh_ref[...], preferred_element_type=jnp.float32)
    gh = gh + bhh_ref[...]
    r = jax.nn.sigmoid(gi[:, :H] + gh[:, :H])
    zg = jax.nn.sigmoid(gi[:, H:2 * H] + gh[:, H:2 * H])
    n = jnp.tanh(gi[:, 2 * H:] + r * gh[:, 2 * H:])
    return (1.0 - zg) * n + zg * h


def _gru_msg_body(parts_ref, h_ref, wih_ref, whh_ref, bih_ref, bhh_ref,
                  wn_ref, hn_ref, mn_ref):
    hn = _gru_core(parts_ref, h_ref, wih_ref, whh_ref, bih_ref, bhh_ref)
    hn_ref[...] = hn
    mn_ref[...] = jnp.dot(hn, wn_ref[...], preferred_element_type=jnp.float32)


def _gru_out_body(parts_ref, h_ref, wih_ref, whh_ref, bih_ref, bhh_ref,
                  w2_ref, b2_ref, out_ref):
    hn = _gru_core(parts_ref, h_ref, wih_ref, whh_ref, bih_ref, bhh_ref)
    hn = jnp.maximum(hn, 0.0)
    out = jnp.dot(hn, w2_ref[...], preferred_element_type=jnp.float32)
    out_ref[...] = out + b2_ref[...]


def _row_spec(cols):
    return pl.BlockSpec((ROW_BLK, cols), lambda r: (r, 0))


def _full_spec(rows, cols):
    return pl.BlockSpec((rows, cols), lambda r: (0, 0))


def _parts_spec():
    return pl.BlockSpec((NC, ROW_BLK, H), lambda r: (0, r, 0))


_GRID = (N // ROW_BLK,)


@jax.jit
def _lin_msg(x, lin_W, lin_b, w0):
    return pl.pallas_call(
        _lin_msg_body,
        grid=_GRID,
        in_specs=[
            _row_spec(H),
            _full_spec(H, H),
            _full_spec(1, H),
            _full_spec(H, H),
        ],
        out_specs=[_row_spec(H), _row_spec(H)],
        out_shape=[
            jax.ShapeDtypeStruct((N, H), jnp.float32),
            jax.ShapeDtypeStruct((N, H), jnp.float32),
        ],
    )(x, lin_W, lin_b, w0)


@jax.jit
def _gru_msg(parts, h, wihT, whhT, bih, bhh, wn):
    return pl.pallas_call(
        _gru_msg_body,
        grid=_GRID,
        in_specs=[
            _parts_spec(),
            _row_spec(H),
            _full_spec(H, 3 * H),
            _full_spec(H, 3 * H),
            _full_spec(1, 3 * H),
            _full_spec(1, 3 * H),
            _full_spec(H, H),
        ],
        out_specs=[_row_spec(H), _row_spec(H)],
        out_shape=[
            jax.ShapeDtypeStruct((N, H), jnp.float32),
            jax.ShapeDtypeStruct((N, H), jnp.float32),
        ],
    )(parts, h, wihT, whhT, bih, bhh, wn)


@jax.jit
def _gru_out(parts, h, wihT, whhT, bih, bhh, w2, b2):
    return pl.pallas_call(
        _gru_out_body,
        grid=_GRID,
        in_specs=[
            _parts_spec(),
            _row_spec(H),
            _full_spec(H, 3 * H),
            _full_spec(H, 3 * H),
            _full_spec(1, 3 * H),
            _full_spec(1, 3 * H),
            _full_spec(H, H),
            _full_spec(1, H),
        ],
        out_specs=_row_spec(H),
        out_shape=jax.ShapeDtypeStruct((N, H), jnp.float32),
    )(parts, h, wihT, whhT, bih, bhh, w2, b2)


def kernel(x, edge_index, lin_W, lin_b, ggc_weight, W_ih, W_hh, b_ih, b_hh,
           lin2_W, lin2_b):
    E = edge_index.shape[1]
    n_chunks = -(-E // (NW * K))
    n_chunks = -(-n_chunks // 2) * 2
    e_pad = NW * n_chunks * K

    src = edge_index[0].astype(jnp.int32)
    dst = edge_index[1].astype(jnp.int32)
    if e_pad != E:
        pad = e_pad - E
        ar = jnp.arange(pad, dtype=jnp.int32)
        src = jnp.concatenate([src, ar % N])
        dst = jnp.concatenate([dst, N + ar % (N_PAD - N)])
    src3 = src.reshape(NW, n_chunks, K)
    dst3 = dst.reshape(NW, n_chunks, K)

    sc_scatter = _make_sc_scatter(n_chunks)

    wihT = W_ih.T
    whhT = W_hh.T
    bih = b_ih.reshape(1, 3 * H)
    bhh = b_hh.reshape(1, 3 * H)
    lb = lin_b.reshape(1, H)
    b2 = lin2_b.reshape(1, H)

    h, m = _lin_msg(x, lin_W, lb, ggc_weight[0])
    for i in range(L):
        parts = sc_scatter(src3, dst3, m)
        if i < L - 1:
            h, m = _gru_msg(parts, h, wihT, whhT, bih, bhh, ggc_weight[i + 1])
        else:
            out = _gru_out(parts, h, wihT, whhT, bih, bhh, lin2_W, b2)
    return out


# speedup vs baseline: 3.2564x; 1.1429x over previous
import functools

import jax
import jax.numpy as jnp
from jax import lax
from jax.experimental import pallas as pl
from jax.experimental.pallas import tpu as pltpu, tpu_sc as plsc

N = 10000
H = 128
L = 3

NC = 2
NS = 16
NW = NC * NS
K = 128
NBUF = 2

N_PAD = 10112
RPT = N_PAD // NS

ROW_BLK = 1000


def _sc_scatter_body(src_hbm, dst_hbm, m_hbm, out_hbm, src_v, dst_v, buf_a,
                     buf_b, agg_sh, sem_a, sem_b):
    c = lax.axis_index("c")
    s = lax.axis_index("s")
    w = c * NS + s
    n_chunks = src_hbm.shape[1]
    c2 = n_chunks // 2

    z = jnp.zeros((16,), jnp.float32)

    def zrow(i, carry):
        for j in range(H // 16):
            buf_a[i, pl.ds(j * 16, 16)] = z
        return carry

    lax.fori_loop(0, K, zrow, 0)
    r0 = s * RPT
    full, rem = RPT // K, RPT % K
    for t in range(full):
        pltpu.sync_copy(buf_a, agg_sh.at[pl.ds(r0 + t * K, K)])
    if rem:
        pltpu.sync_copy(buf_a.at[pl.ds(0, rem)],
                        agg_sh.at[pl.ds(r0 + full * K, rem)])
    plsc.subcore_barrier()

    def run_half(h0):
        pltpu.sync_copy(src_hbm.at[w, pl.ds(h0, c2)], src_v)
        pltpu.sync_copy(dst_hbm.at[w, pl.ds(h0, c2)], dst_v)

        def pair(g, carry):
            j0 = 2 * g
            j1 = 2 * g + 1
            da = pltpu.async_copy(m_hbm.at[src_v.at[j0]], buf_a, sem_a)
            db = pltpu.async_copy(m_hbm.at[src_v.at[j1]], buf_b, sem_b)
            da.wait()
            pltpu.sync_copy(buf_a, agg_sh.at[dst_v.at[j0]], add=True)
            db.wait()
            pltpu.sync_copy(buf_b, agg_sh.at[dst_v.at[j1]], add=True)
            return carry

        lax.fori_loop(0, c2 // 2, pair, 0)

    run_half(0)
    run_half(c2)
    plsc.subcore_barrier()

    pltpu.sync_copy(agg_sh.at[pl.ds(r0, RPT)], out_hbm.at[c, pl.ds(r0, RPT)])


def _make_sc_scatter(n_chunks):
    mesh = plsc.VectorSubcoreMesh(core_axis_name="c", subcore_axis_name="s",
                                  num_cores=NC, num_subcores=NS)

    return pl.kernel(
        _sc_scatter_body,
        out_type=jax.ShapeDtypeStruct((NC, N_PAD, H), jnp.float32),
        mesh=mesh,
        scratch_types=[
            pltpu.VMEM((n_chunks // 2, K), jnp.int32),
            pltpu.VMEM((n_chunks // 2, K), jnp.int32),
            pltpu.VMEM((K, H), jnp.float32),
            pltpu.VMEM((K, H), jnp.float32),
            pltpu.VMEM_SHARED((N_PAD, H), jnp.float32),
            pltpu.SemaphoreType.DMA,
            pltpu.SemaphoreType.DMA,
        ],
    )


def _lin_msg_body(x_ref, lw_ref, lb_ref, w0_ref, h_ref, m_ref):
    h = jnp.dot(x_ref[...], lw_ref[...], preferred_element_type=jnp.float32)
    h = h + lb_ref[...]
    h_ref[...] = h
    m_ref[...] = jnp.dot(h, w0_ref[...], preferred_element_type=jnp.float32)


def _gru_core(parts_ref, h_ref, wih_ref, whh_ref, bih_ref, bhh_ref):
    agg = parts_ref[0] + parts_ref[1]
    h = h_ref[...]
    gi = jnp.dot(agg, wih_ref[...], preferred_element_type=jnp.float32)
    gi = gi + bih_ref[...]
    gh = jnp.dot(h, whh_ref[...], preferred_element_type=jnp.float32)
    gh = gh + bhh_ref[...]
    r = jax.nn.sigmoid(gi[:, :H] + gh[:, :H])
    zg = jax.nn.sigmoid(gi[:, H:2 * H] + gh[:, H:2 * H])
    n = jnp.tanh(gi[:, 2 * H:] + r * gh[:, 2 * H:])
    return (1.0 - zg) * n + zg * h


def _gru_msg_body(parts_ref, h_ref, wih_ref, whh_ref, bih_ref, bhh_ref,
                  wn_ref, hn_ref, mn_ref):
    hn = _gru_core(parts_ref, h_ref, wih_ref, whh_ref, bih_ref, bhh_ref)
    hn_ref[...] = hn
    mn_ref[...] = jnp.dot(hn, wn_ref[...], preferred_element_type=jnp.float32)


def _gru_out_body(parts_ref, h_ref, wih_ref, whh_ref, bih_ref, bhh_ref,
                  w2_ref, b2_ref, out_ref):
    hn = _gru_core(parts_ref, h_ref, wih_ref, whh_ref, bih_ref, bhh_ref)
    hn = jnp.maximum(hn, 0.0)
    out = jnp.dot(hn, w2_ref[...], preferred_element_type=jnp.float32)
    out_ref[...] = out + b2_ref[...]


def _row_spec(cols):
    return pl.BlockSpec((ROW_BLK, cols), lambda r: (r, 0))


def _full_spec(rows, cols):
    return pl.BlockSpec((rows, cols), lambda r: (0, 0))


def _parts_spec():
    return pl.BlockSpec((NC, ROW_BLK, H), lambda r: (0, r, 0))


_GRID = (N // ROW_BLK,)


@jax.jit
def _lin_msg(x, lin_W, lin_b, w0):
    return pl.pallas_call(
        _lin_msg_body,
        grid=_GRID,
        in_specs=[
            _row_spec(H),
            _full_spec(H, H),
            _full_spec(1, H),
            _full_spec(H, H),
        ],
        out_specs=[_row_spec(H), _row_spec(H)],
        out_shape=[
            jax.ShapeDtypeStruct((N, H), jnp.float32),
            jax.ShapeDtypeStruct((N, H), jnp.float32),
        ],
    )(x, lin_W, lin_b, w0)


@jax.jit
def _gru_msg(parts, h, wihT, whhT, bih, bhh, wn):
    return pl.pallas_call(
        _gru_msg_body,
        grid=_GRID,
        in_specs=[
            _parts_spec(),
            _row_spec(H),
            _full_spec(H, 3 * H),
            _full_spec(H, 3 * H),
            _full_spec(1, 3 * H),
            _full_spec(1, 3 * H),
            _full_spec(H, H),
        ],
        out_specs=[_row_spec(H), _row_spec(H)],
        out_shape=[
            jax.ShapeDtypeStruct((N, H), jnp.float32),
            jax.ShapeDtypeStruct((N, H), jnp.float32),
        ],
    )(parts, h, wihT, whhT, bih, bhh, wn)


@jax.jit
def _gru_out(parts, h, wihT, whhT, bih, bhh, w2, b2):
    return pl.pallas_call(
        _gru_out_body,
        grid=_GRID,
        in_specs=[
            _parts_spec(),
            _row_spec(H),
            _full_spec(H, 3 * H),
            _full_spec(H, 3 * H),
            _full_spec(1, 3 * H),
            _full_spec(1, 3 * H),
            _full_spec(H, H),
            _full_spec(1, H),
        ],
        out_specs=_row_spec(H),
        out_shape=jax.ShapeDtypeStruct((N, H), jnp.float32),
    )(parts, h, wihT, whhT, bih, bhh, w2, b2)


def kernel(x, edge_index, lin_W, lin_b, ggc_weight, W_ih, W_hh, b_ih, b_hh,
           lin2_W, lin2_b):
    E = edge_index.shape[1]
    n_chunks = -(-E // (NW * K))
    n_chunks = -(-n_chunks // 4) * 4
    e_pad = NW * n_chunks * K

    src = edge_index[0].astype(jnp.int32)
    dst = edge_index[1].astype(jnp.int32)
    if e_pad != E:
        pad = e_pad - E
        ar = jnp.arange(pad, dtype=jnp.int32)
        src = jnp.concatenate([src, ar % N])
        dst = jnp.concatenate([dst, N + ar % (N_PAD - N)])
    src3 = src.reshape(NW, n_chunks, K)
    dst3 = dst.reshape(NW, n_chunks, K)

    sc_scatter = _make_sc_scatter(n_chunks)

    wihT = W_ih.T
    whhT = W_hh.T
    bih = b_ih.reshape(1, 3 * H)
    bhh = b_hh.reshape(1, 3 * H)
    lb = lin_b.reshape(1, H)
    b2 = lin2_b.reshape(1, H)

    h, m = _lin_msg(x, lin_W, lb, ggc_weight[0])
    for i in range(L):
        parts = sc_scatter(src3, dst3, m)
        if i < L - 1:
            h, m = _gru_msg(parts, h, wihT, whhT, bih, bhh, ggc_weight[i + 1])
        else:
            out = _gru_out(parts, h, wihT, whhT, bih, bhh, lin2_W, b2)
    return out


# speedup vs baseline: 4.2190x; 1.2956x over previous
import functools

import jax
import jax.numpy as jnp
from jax import lax
from jax.experimental import pallas as pl
from jax.experimental.pallas import tpu as pltpu, tpu_sc as plsc

N = 10000
H = 128
L = 3

NC = 2
NS = 16
NW = NC * NS
K = 128
NBUF = 2

N_PAD = 10112
RPT = N_PAD // NS

ROW_BLK = 1000


def _sc_scatter_body(src_hbm, dst_hbm, m_hbm, out_hbm, src_v, dst_v, buf_a,
                     buf_b, agg_sh, sem_a, sem_b):
    c = lax.axis_index("c")
    s = lax.axis_index("s")
    w = c * NS + s
    n_chunks = src_hbm.shape[1]
    c2 = n_chunks // 2

    z = jnp.zeros((16,), jnp.float32)

    def zrow(i, carry):
        for j in range(H // 16):
            buf_a[i, pl.ds(j * 16, 16)] = z
        return carry

    lax.fori_loop(0, K, zrow, 0)
    r0 = s * RPT
    full, rem = RPT // K, RPT % K
    for t in range(full):
        pltpu.sync_copy(buf_a, agg_sh.at[pl.ds(r0 + t * K, K)])
    if rem:
        pltpu.sync_copy(buf_a.at[pl.ds(0, rem)],
                        agg_sh.at[pl.ds(r0 + full * K, rem)])
    plsc.subcore_barrier()

    def run_half(h0):
        pltpu.sync_copy(src_hbm.at[w, pl.ds(h0, c2)], src_v)
        pltpu.sync_copy(dst_hbm.at[w, pl.ds(h0, c2)], dst_v)

        pltpu.async_copy(m_hbm.at[src_v.at[0]], buf_a, sem_a)
        pltpu.async_copy(m_hbm.at[src_v.at[1]], buf_b, sem_b)

        def pair(g, carry):
            j0 = 2 * g
            j1 = 2 * g + 1
            pltpu.make_async_copy(m_hbm.at[src_v.at[j0]], buf_a,
                                  sem_a).wait()
            pltpu.sync_copy(buf_a, agg_sh.at[dst_v.at[j0]], add=True)

            @pl.when(j0 + 2 < c2)
            def _():
                pltpu.async_copy(m_hbm.at[src_v.at[j0 + 2]], buf_a, sem_a)

            pltpu.make_async_copy(m_hbm.at[src_v.at[j1]], buf_b,
                                  sem_b).wait()
            pltpu.sync_copy(buf_b, agg_sh.at[dst_v.at[j1]], add=True)

            @pl.when(j1 + 2 < c2)
            def _():
                pltpu.async_copy(m_hbm.at[src_v.at[j1 + 2]], buf_b, sem_b)

            return carry

        lax.fori_loop(0, c2 // 2, pair, 0)

    run_half(0)
    run_half(c2)
    plsc.subcore_barrier()

    pltpu.sync_copy(agg_sh.at[pl.ds(r0, RPT)], out_hbm.at[c, pl.ds(r0, RPT)])


def _make_sc_scatter(n_chunks):
    mesh = plsc.VectorSubcoreMesh(core_axis_name="c", subcore_axis_name="s",
                                  num_cores=NC, num_subcores=NS)

    return pl.kernel(
        _sc_scatter_body,
        out_type=jax.ShapeDtypeStruct((NC, N_PAD, H), jnp.float32),
        mesh=mesh,
        scratch_types=[
            pltpu.VMEM((n_chunks // 2, K), jnp.int32),
            pltpu.VMEM((n_chunks // 2, K), jnp.int32),
            pltpu.VMEM((K, H), jnp.float32),
            pltpu.VMEM((K, H), jnp.float32),
            pltpu.VMEM_SHARED((N_PAD, H), jnp.float32),
            pltpu.SemaphoreType.DMA,
            pltpu.SemaphoreType.DMA,
        ],
    )


def _lin_msg_body(x_ref, lw_ref, lb_ref, w0_ref, h_ref, m_ref):
    h = jnp.dot(x_ref[...], lw_ref[...], preferred_element_type=jnp.float32)
    h = h + lb_ref[...]
    h_ref[...] = h
    m_ref[...] = jnp.dot(h, w0_ref[...], preferred_element_type=jnp.float32)


def _gru_core(parts_ref, h_ref, wih_ref, whh_ref, bih_ref, bhh_ref):
    agg = parts_ref[0] + parts_ref[1]
    h = h_ref[...]
    gi = jnp.dot(agg, wih_ref[...], preferred_element_type=jnp.float32)
    gi = gi + bih_ref[...]
    gh = jnp.dot(h, whh_ref[...], preferred_element_type=jnp.float32)
    gh = gh + bhh_ref[...]
    r = jax.nn.sigmoid(gi[:, :H] + gh[:, :H])
    zg = jax.nn.sigmoid(gi[:, H:2 * H] + gh[:, H:2 * H])
    n = jnp.tanh(gi[:, 2 * H:] + r * gh[:, 2 * H:])
    return (1.0 - zg) * n + zg * h


def _gru_msg_body(parts_ref, h_ref, wih_ref, whh_ref, bih_ref, bhh_ref,
                  wn_ref, hn_ref, mn_ref):
    hn = _gru_core(parts_ref, h_ref, wih_ref, whh_ref, bih_ref, bhh_ref)
    hn_ref[...] = hn
    mn_ref[...] = jnp.dot(hn, wn_ref[...], preferred_element_type=jnp.float32)


def _gru_out_body(parts_ref, h_ref, wih_ref, whh_ref, bih_ref, bhh_ref,
                  w2_ref, b2_ref, out_ref):
    hn = _gru_core(parts_ref, h_ref, wih_ref, whh_ref, bih_ref, bhh_ref)
    hn = jnp.maximum(hn, 0.0)
    out = jnp.dot(hn, w2_ref[...], preferred_element_type=jnp.float32)
    out_ref[...] = out + b2_ref[...]


def _row_spec(cols):
    return pl.BlockSpec((ROW_BLK, cols), lambda r: (r, 0))


def _full_spec(rows, cols):
    return pl.BlockSpec((rows, cols), lambda r: (0, 0))


def _parts_spec():
    return pl.BlockSpec((NC, ROW_BLK, H), lambda r: (0, r, 0))


_GRID = (N // ROW_BLK,)


@jax.jit
def _lin_msg(x, lin_W, lin_b, w0):
    return pl.pallas_call(
        _lin_msg_body,
        grid=_GRID,
        in_specs=[
            _row_spec(H),
            _full_spec(H, H),
            _full_spec(1, H),
            _full_spec(H, H),
        ],
        out_specs=[_row_spec(H), _row_spec(H)],
        out_shape=[
            jax.ShapeDtypeStruct((N, H), jnp.float32),
            jax.ShapeDtypeStruct((N, H), jnp.float32),
        ],
    )(x, lin_W, lin_b, w0)


@jax.jit
def _gru_msg(parts, h, wihT, whhT, bih, bhh, wn):
    return pl.pallas_call(
        _gru_msg_body,
        grid=_GRID,
        in_specs=[
            _parts_spec(),
            _row_spec(H),
            _full_spec(H, 3 * H),
            _full_spec(H, 3 * H),
            _full_spec(1, 3 * H),
            _full_spec(1, 3 * H),
            _full_spec(H, H),
        ],
        out_specs=[_row_spec(H), _row_spec(H)],
        out_shape=[
            jax.ShapeDtypeStruct((N, H), jnp.float32),
            jax.ShapeDtypeStruct((N, H), jnp.float32),
        ],
    )(parts, h, wihT, whhT, bih, bhh, wn)


@jax.jit
def _gru_out(parts, h, wihT, whhT, bih, bhh, w2, b2):
    return pl.pallas_call(
        _gru_out_body,
        grid=_GRID,
        in_specs=[
            _parts_spec(),
            _row_spec(H),
            _full_spec(H, 3 * H),
            _full_spec(H, 3 * H),
            _full_spec(1, 3 * H),
            _full_spec(1, 3 * H),
            _full_spec(H, H),
            _full_spec(1, H),
        ],
        out_specs=_row_spec(H),
        out_shape=jax.ShapeDtypeStruct((N, H), jnp.float32),
    )(parts, h, wihT, whhT, bih, bhh, w2, b2)


def kernel(x, edge_index, lin_W, lin_b, ggc_weight, W_ih, W_hh, b_ih, b_hh,
           lin2_W, lin2_b):
    E = edge_index.shape[1]
    n_chunks = -(-E // (NW * K))
    n_chunks = -(-n_chunks // 4) * 4
    e_pad = NW * n_chunks * K

    src = edge_index[0].astype(jnp.int32)
    dst = edge_index[1].astype(jnp.int32)
    if e_pad != E:
        pad = e_pad - E
        ar = jnp.arange(pad, dtype=jnp.int32)
        src = jnp.concatenate([src, ar % N])
        dst = jnp.concatenate([dst, N + ar % (N_PAD - N)])
    src3 = src.reshape(NW, n_chunks, K)
    dst3 = dst.reshape(NW, n_chunks, K)

    sc_scatter = _make_sc_scatter(n_chunks)

    wihT = W_ih.T
    whhT = W_hh.T
    bih = b_ih.reshape(1, 3 * H)
    bhh = b_hh.reshape(1, 3 * H)
    lb = lin_b.reshape(1, H)
    b2 = lin2_b.reshape(1, H)

    h, m = _lin_msg(x, lin_W, lb, ggc_weight[0])
    for i in range(L):
        parts = sc_scatter(src3, dst3, m)
        if i < L - 1:
            h, m = _gru_msg(parts, h, wihT, whhT, bih, bhh, ggc_weight[i + 1])
        else:
            out = _gru_out(parts, h, wihT, whhT, bih, bhh, lin2_W, b2)
    return out
